# Initial kernel scaffold; baseline (speedup 1.0000x reference)
import jax
import jax.numpy as jnp
from jax.experimental import pallas as pl


def kernel(x, edge_index, batch, W0_1, W1_1, b1, W0_2, W1_2, b2, W0_3, W1_3, b3, Wl, bl):
    raise NotImplementedError("write your pallas kernel here")


# speedup vs baseline: 7.5243x; 7.5243x over previous
import functools

import jax
import jax.numpy as jnp
from jax import lax
from jax.experimental import pallas as pl
from jax.experimental.pallas import tpu as pltpu
from jax.experimental.pallas import tpu_sc as plsc

NC = 2
NS = 16
NW = NC * NS
K = 128
G = 64


def _pad_edges(edge_index, n):
  e = edge_index.shape[1]
  epw = -(-e // (NW * K)) * K
  epad = NW * epw
  pad = epad - e
  row = edge_index[0]
  col = edge_index[1]
  trash = n + (jnp.arange(pad, dtype=jnp.int32) % 8)
  row_g = jnp.concatenate([row, jnp.zeros((pad,), jnp.int32)])
  col_s = jnp.concatenate([col, trash])
  row_s = jnp.concatenate([row, trash])
  nchunk = epw // K
  return (row_g.reshape(NW, nchunk, K), col_s.reshape(NW, nchunk, K),
          row_s.reshape(NW, nchunk, K), nchunk)


def _acc_rows(n):
  return -(-(n + 8) // (128 * NS)) * (128 * NS)


def _wsplit(n):
  wa = -(-(-(-n // NS)) // 8) * 8
  wl = n - (NS - 1) * wa
  assert wl > 0
  return wa, wl


def _make_deg_kernel(n, nchunk):
  na = _acc_rows(n)
  nzt = na // (128 * NS)
  wa, wl = _wsplit(n)
  d = 128

  @functools.partial(
      pl.kernel,
      out_type=jax.ShapeDtypeStruct((NC, n, d), jnp.float32),
      mesh=plsc.VectorSubcoreMesh(core_axis_name="c", subcore_axis_name="s"),
      scratch_types=[
          pltpu.VMEM((nchunk, K), jnp.int32),
          pltpu.VMEM((K, d), jnp.float32),
          pltpu.VMEM_SHARED((na, d), jnp.float32),
      ],
  )
  def deg_kernel(rows_hbm, out_hbm, rbuf, ones, acc):
    c = lax.axis_index("c")
    s = lax.axis_index("s")
    wid = s * NC + c

    nv = d // 16

    def fill_zero(i, _):
      ones[i // nv, pl.ds((i % nv) * 16, 16)] = jnp.zeros((16,), jnp.float32)
      return 0
    lax.fori_loop(0, K * nv, fill_zero, 0)
    for j in range(nzt):
      pltpu.sync_copy(ones, acc.at[pl.ds((s * nzt + j) * 128, 128)])
    plsc.subcore_barrier()

    def fill_ones(i, _):
      ones[i // nv, pl.ds((i % nv) * 16, 16)] = jnp.ones((16,), jnp.float32)
      return 0
    lax.fori_loop(0, K * nv, fill_ones, 0)

    pltpu.sync_copy(rows_hbm.at[wid], rbuf)

    def body(j, _):
      pltpu.sync_copy(ones, acc.at[rbuf.at[j]], add=True)
      return 0
    lax.fori_loop(0, nchunk, body, 0)
    plsc.subcore_barrier()

    @pl.when(s < NS - 1)
    def _():
      pltpu.sync_copy(acc.at[pl.ds(s * wa, wa)],
                      out_hbm.at[c, pl.ds(s * wa, wa)])

    @pl.when(s == NS - 1)
    def _():
      pltpu.sync_copy(acc.at[pl.ds((NS - 1) * wa, wl)],
                      out_hbm.at[c, pl.ds((NS - 1) * wa, wl)])

  return deg_kernel


def _make_scatter_kernel(n, d, nchunk):
  na = _acc_rows(n)
  nzt = na // (128 * NS)
  wa, wl = _wsplit(n)

  @functools.partial(
      pl.kernel,
      out_type=jax.ShapeDtypeStruct((NC, n, d), jnp.float32),
      mesh=plsc.VectorSubcoreMesh(core_axis_name="c", subcore_axis_name="s"),
      scratch_types=[
          pltpu.VMEM((nchunk, K), jnp.int32),
          pltpu.VMEM((nchunk, K), jnp.int32),
          pltpu.VMEM((K, d), jnp.float32),
          pltpu.VMEM_SHARED((na, d), jnp.float32),
          pltpu.SemaphoreType.DMA,
      ],
  )
  def scatter_kernel(rows_hbm, cols_hbm, y_hbm, out_hbm,
                     rbuf, cbuf, rows, acc, sem):
    c = lax.axis_index("c")
    s = lax.axis_index("s")
    wid = s * NC + c

    nv = d // 16

    def fill_zero(i, _):
      rows[i // nv, pl.ds((i % nv) * 16, 16)] = jnp.zeros((16,), jnp.float32)
      return 0
    lax.fori_loop(0, K * nv, fill_zero, 0)
    for j in range(nzt):
      pltpu.sync_copy(rows, acc.at[pl.ds((s * nzt + j) * 128, 128)])
    plsc.subcore_barrier()

    pltpu.sync_copy(rows_hbm.at[wid], rbuf)
    pltpu.sync_copy(cols_hbm.at[wid], cbuf)

    def body(j, _):
      pltpu.async_copy(y_hbm.at[rbuf.at[j]], rows, sem).wait()
      pltpu.sync_copy(rows, acc.at[cbuf.at[j]], add=True)
      return 0
    lax.fori_loop(0, nchunk, body, 0)
    plsc.subcore_barrier()

    @pl.when(s < NS - 1)
    def _():
      pltpu.sync_copy(acc.at[pl.ds(s * wa, wa)],
                      out_hbm.at[c, pl.ds(s * wa, wa)])

    @pl.when(s == NS - 1)
    def _():
      pltpu.sync_copy(acc.at[pl.ds((NS - 1) * wa, wl)],
                      out_hbm.at[c, pl.ds((NS - 1) * wa, wl)])

  return scatter_kernel


def _tc_prescale(degp, x):
  n, d = x.shape

  def body(degp_ref, x_ref, dis_ref, y_ref):
    deg = degp_ref[0] + degp_ref[1]
    dis = jnp.where(deg > 0, lax.rsqrt(jnp.maximum(deg, 1e-30)), 0.0)
    dis_ref[...] = dis
    y_ref[...] = dis * x_ref[...]

  return pl.pallas_call(
      body,
      out_shape=[jax.ShapeDtypeStruct((n, 1), jnp.float32),
                 jax.ShapeDtypeStruct((n, d), jnp.float32)],
  )(degp, x)


def _tc_layer(h, p, dis, w0, w1, b, relu, want_y):
  n, d = h.shape
  hh = w0.shape[1]

  def body(h_ref, p_ref, dis_ref, w0_ref, w1_ref, b_ref, *outs):
    t = dis_ref[...] * (p_ref[0] + p_ref[1])
    z = (jnp.dot(h_ref[...], w0_ref[...], preferred_element_type=jnp.float32)
         - jnp.dot(t, w1_ref[...], preferred_element_type=jnp.float32)
         + b_ref[...][None, :])
    if relu:
      z = jnp.maximum(z, 0.0)
    outs[0][...] = z
    if want_y:
      outs[1][...] = dis_ref[...] * z

  out_shape = [jax.ShapeDtypeStruct((n, hh), jnp.float32)]
  if want_y:
    out_shape.append(jax.ShapeDtypeStruct((n, hh), jnp.float32))
  return pl.pallas_call(body, out_shape=out_shape)(h, p, dis, w0, w1, b)


def _tc_final(h, p, dis, w0, w1, b, batch2, wl, bl):
  n, d = h.shape
  hh = w0.shape[1]
  co = wl.shape[1]

  def body(h_ref, p_ref, dis_ref, w0_ref, w1_ref, b_ref, batch_ref,
           wl_ref, bl_ref, out_ref):
    t = dis_ref[...] * (p_ref[0] + p_ref[1])
    h3 = (jnp.dot(h_ref[...], w0_ref[...], preferred_element_type=jnp.float32)
          - jnp.dot(t, w1_ref[...], preferred_element_type=jnp.float32)
          + b_ref[...][None, :])
    seg = lax.broadcasted_iota(jnp.int32, (G, n), 0)
    m = (batch_ref[...] == seg).astype(jnp.float32)
    sums = jnp.dot(m, h3, preferred_element_type=jnp.float32)
    counts = jnp.sum(m, axis=1, keepdims=True)
    pooled = sums / jnp.maximum(counts, 1.0)
    out_ref[...] = (jnp.dot(pooled, wl_ref[...],
                            preferred_element_type=jnp.float32)
                    + bl_ref[...][None, :])

  return pl.pallas_call(
      body,
      out_shape=jax.ShapeDtypeStruct((G, co), jnp.float32),
  )(h, p, dis, w0, w1, b, batch2, wl, bl)


def kernel(x, edge_index, batch, W0_1, W1_1, b1, W0_2, W1_2, b2,
           W0_3, W1_3, b3, Wl, bl):
  n, d = x.shape
  row_g, col_s, row_s, nchunk = _pad_edges(edge_index, n)

  deg_kernel = _make_deg_kernel(n, nchunk)
  scat = _make_scatter_kernel(n, d, nchunk)

  degp = deg_kernel(row_s)
  dis, y0 = _tc_prescale(degp[:, :, 0:1], x)

  p1 = scat(row_g, col_s, y0)
  h1, y1 = _tc_layer(x, p1, dis, W0_1, W1_1, b1, relu=True, want_y=True)

  p2 = scat(row_g, col_s, y1)
  h2, y2 = _tc_layer(h1, p2, dis, W0_2, W1_2, b2, relu=True, want_y=True)

  p3 = scat(row_g, col_s, y2)
  batch2 = batch.reshape(1, n).astype(jnp.int32)
  return _tc_final(h2, p3, dis, W0_3, W1_3, b3, batch2, Wl, bl)
